# baseline (speedup 1.0000x reference)
import functools

import jax
import jax.numpy as jnp
from jax import lax
from jax.experimental import pallas as pl
from jax.experimental.pallas import tpu as pltpu
from jax.experimental.pallas import tpu_sc as plsc

SZ = 2048
TEMP = 10.0
EPS = 1e-20
LN2 = 0.6931471805599453

SC_ROWS = 1152
TC_ROWS_PER_BLOCK = 64

_NC, _NS = 2, 16
_NW = _NC * _NS


def _native_view(x):
    return (
        x.reshape(SZ, 16, 128, 2)
        .transpose(0, 1, 3, 2)
        .reshape(SZ, 32, 128)
    )


def _tc_body(g_ref, u_ref, o_ref):
    for g in range(16):
        a0 = g_ref[:, 2 * g, :]
        a1 = g_ref[:, 2 * g + 1, :]
        u0 = u_ref[:, 2 * g, :]
        u1 = u_ref[:, 2 * g + 1, :]
        L0 = EPS - jnp.log(u0 + EPS)
        L1 = EPS - jnp.log(u1 + EPS)
        lr = jnp.log(L0 / L1)
        s = (a0 - a1 - lr) * (1.0 / TEMP)
        o_ref[:, 128 * g:128 * (g + 1)] = 1.0 / (1.0 + jnp.exp(-s))


def _tc_call(gm, uu, row0, nrows, full=False):
    grid = nrows // TC_ROWS_PER_BLOCK
    off = row0 // TC_ROWS_PER_BLOCK
    oshape = (SZ, SZ) if full else (nrows, SZ)
    oidx = (lambda i: (i + off, 0)) if full else (lambda i: (i, 0))
    return pl.pallas_call(
        _tc_body,
        grid=(grid,),
        in_specs=[
            pl.BlockSpec((TC_ROWS_PER_BLOCK, 32, 128), lambda i: (i + off, 0, 0)),
            pl.BlockSpec((TC_ROWS_PER_BLOCK, 32, 128), lambda i: (i + off, 0, 0)),
        ],
        out_specs=pl.BlockSpec((TC_ROWS_PER_BLOCK, SZ), oidx),
        out_shape=jax.ShapeDtypeStruct(oshape, jnp.float32),
    )(gm, uu)


_C = 4
_NBUF = 2


_P8 = (0.9999936302584942, -0.49982549864297415, 0.33144665223895153,
       -0.23943337072679435, 0.16499812978978992, -0.09229041732432854,
       0.03426459991863622, -0.006006605041074868)
_P5 = (0.9991664010110767, -0.48969909032090764, 0.2838231830655365,
       -0.12995719765851482, 0.029808765243560027)


def _ln_poly(v, coeffs):
    b = lax.bitcast_convert_type(v, jnp.int32)
    e = lax.shift_right_arithmetic(b, 23) - 127
    m = lax.bitcast_convert_type(
        (b & 0x007FFFFF) | 0x3F800000, jnp.float32)
    t = m - 1.0
    p = coeffs[-1]
    for c in reversed(coeffs[:-1]):
        p = c + t * p
    return e.astype(jnp.float32) * LN2 + t * p


def _sc_body(nrows, g_hbm, u_hbm, o_hbm, gbuf, ubuf, obuf, gsem, usem, osem):
    wid = lax.axis_index("s") * _NC + lax.axis_index("c")
    rpw = nrows // _NW
    base = wid * rpw
    nchunks = rpw // _C

    def issue_in(k):
        slot = lax.rem(k, _NBUF)
        row = base + k * _C
        pltpu.async_copy(g_hbm.at[pl.ds(row, _C)], gbuf.at[slot], gsem)
        pltpu.async_copy(u_hbm.at[pl.ds(row, _C)], ubuf.at[slot], usem)

    def wait_in():
        pltpu.make_async_copy(g_hbm.at[pl.ds(0, _C)], gbuf.at[0], gsem).wait()
        pltpu.make_async_copy(u_hbm.at[pl.ds(0, _C)], ubuf.at[0], usem).wait()

    def issue_out(k):
        slot = lax.rem(k, _NBUF)
        row = base + k * _C
        pltpu.async_copy(obuf.at[slot], o_hbm.at[pl.ds(row, _C)], osem)

    def wait_out():
        pltpu.make_async_copy(obuf.at[0], o_hbm.at[pl.ds(0, _C)], osem).wait()

    def compute(k):
        slot = lax.rem(k, _NBUF)

        @plsc.parallel_loop(0, _C * 128, 1, unroll=8)
        def _(q):
            rc = lax.shift_right_logical(q, 7)
            gg = q & 127
            g2 = lax.shift_right_logical(gg, 3) * 2
            col = (gg & 7) * 16
            a0 = gbuf[slot, rc, g2, pl.ds(col, 16)]
            a1 = gbuf[slot, rc, g2 + 1, pl.ds(col, 16)]
            u0 = ubuf[slot, rc, g2, pl.ds(col, 16)]
            u1 = ubuf[slot, rc, g2 + 1, pl.ds(col, 16)]
            l0 = _ln_poly(u0, _P8)
            l1 = _ln_poly(u1, _P8)
            lr = _ln_poly(l0 / l1, _P5)
            s = (a1 - a0 + lr) * (1.0 / TEMP)
            obuf[slot, rc, pl.ds(gg * 16, 16)] = 1.0 / (1.0 + jnp.exp(s))

    issue_in(0)

    def body(k, c):
        @pl.when(k + 1 < nchunks)
        def _():
            issue_in(k + 1)

        wait_in()

        @pl.when(k >= _NBUF)
        def _():
            wait_out()

        compute(k)
        issue_out(k)
        return c

    lax.fori_loop(0, nchunks, body, 0)
    for _ in range(min(nchunks, _NBUF)):
        wait_out()


def _sc_call(gm, uu, nrows):
    body = functools.partial(_sc_body, nrows)
    mesh = plsc.VectorSubcoreMesh(core_axis_name="c", subcore_axis_name="s")
    fn = pl.kernel(
        body,
        out_type=jax.ShapeDtypeStruct((nrows, SZ), jnp.float32),
        mesh=mesh,
        scratch_types=[
            pltpu.VMEM((_NBUF, _C, 32, 128), jnp.float32),
            pltpu.VMEM((_NBUF, _C, 32, 128), jnp.float32),
            pltpu.VMEM((_NBUF, _C, SZ), jnp.float32),
            pltpu.SemaphoreType.DMA,
            pltpu.SemaphoreType.DMA,
            pltpu.SemaphoreType.DMA,
        ],
    )
    return fn(gm, uu)


def kernel(gen_matrix, u):
    gm = _native_view(gen_matrix.reshape(SZ, SZ, 2))
    uu = _native_view(u.reshape(SZ, SZ, 2))
    if SC_ROWS == 0:
        return _tc_call(gm, uu, 0, SZ)
    if SC_ROWS == SZ:
        return _sc_call(gm, uu, SZ)
    top = _sc_call(gm, uu, SC_ROWS)
    bot = _tc_call(gm, uu, SC_ROWS, SZ - SC_ROWS, full=True)
    return lax.dynamic_update_slice(bot, top, (0, 0))


# speedup vs baseline: 1.0118x; 1.0118x over previous
import functools

import jax
import jax.numpy as jnp
from jax import lax
from jax.experimental import pallas as pl
from jax.experimental.pallas import tpu as pltpu
from jax.experimental.pallas import tpu_sc as plsc

SZ = 2048
TEMP = 10.0
EPS = 1e-20
LN2 = 0.6931471805599453

SC_ROWS = 1152
TC_ROWS_PER_BLOCK = 64

_NC, _NS = 2, 16
_NW = _NC * _NS


def _native_view(x):
    return (
        x.reshape(SZ, 16, 128, 2)
        .transpose(0, 1, 3, 2)
        .reshape(SZ, 32, 128)
    )


def _tc_body(g_ref, u_ref, o_ref):
    for g in range(16):
        a0 = g_ref[:, 2 * g, :]
        a1 = g_ref[:, 2 * g + 1, :]
        u0 = u_ref[:, 2 * g, :]
        u1 = u_ref[:, 2 * g + 1, :]
        L0 = EPS - jnp.log(u0 + EPS)
        L1 = EPS - jnp.log(u1 + EPS)
        lr = jnp.log(L0 / L1)
        s = (a0 - a1 - lr) * (1.0 / TEMP)
        o_ref[:, 128 * g:128 * (g + 1)] = 1.0 / (1.0 + jnp.exp(-s))


def _tc_call(gm, uu, row0, nrows, full=False):
    grid = nrows // TC_ROWS_PER_BLOCK
    off = row0 // TC_ROWS_PER_BLOCK
    oshape = (SZ, SZ) if full else (nrows, SZ)
    oidx = (lambda i: (i + off, 0)) if full else (lambda i: (i, 0))
    return pl.pallas_call(
        _tc_body,
        grid=(grid,),
        in_specs=[
            pl.BlockSpec((TC_ROWS_PER_BLOCK, 32, 128), lambda i: (i + off, 0, 0)),
            pl.BlockSpec((TC_ROWS_PER_BLOCK, 32, 128), lambda i: (i + off, 0, 0)),
        ],
        out_specs=pl.BlockSpec((TC_ROWS_PER_BLOCK, SZ), oidx),
        out_shape=jax.ShapeDtypeStruct(oshape, jnp.float32),
    )(gm, uu)


_C = 4
_NBUF = 2


_P8 = (0.9999936302584942, -0.49982549864297415, 0.33144665223895153,
       -0.23943337072679435, 0.16499812978978992, -0.09229041732432854,
       0.03426459991863622, -0.006006605041074868)
_P5 = (0.9991664010110767, -0.48969909032090764, 0.2838231830655365,
       -0.12995719765851482, 0.029808765243560027)


def _ln_poly(v, coeffs):
    b = lax.bitcast_convert_type(v, jnp.int32)
    e = lax.shift_right_arithmetic(b, 23) - 127
    m = lax.bitcast_convert_type(
        (b & 0x007FFFFF) | 0x3F800000, jnp.float32)
    t = m - 1.0
    p = coeffs[-1]
    for c in reversed(coeffs[:-1]):
        p = c + t * p
    return e.astype(jnp.float32) * LN2 + t * p


def _sc_body(nrows, g_hbm, u_hbm, o_hbm, gbuf, ubuf, obuf, gsem, usem, osem):
    wid = lax.axis_index("s") * _NC + lax.axis_index("c")
    rpw = nrows // _NW
    base = wid * rpw
    nchunks = rpw // _C

    def issue_in(k):
        slot = lax.rem(k, _NBUF)
        row = base + k * _C
        pltpu.async_copy(g_hbm.at[pl.ds(row, _C)], gbuf.at[slot], gsem)
        pltpu.async_copy(u_hbm.at[pl.ds(row, _C)], ubuf.at[slot], usem)

    def wait_in():
        pltpu.make_async_copy(g_hbm.at[pl.ds(0, _C)], gbuf.at[0], gsem).wait()
        pltpu.make_async_copy(u_hbm.at[pl.ds(0, _C)], ubuf.at[0], usem).wait()

    def issue_out(k):
        slot = lax.rem(k, _NBUF)
        row = base + k * _C
        pltpu.async_copy(obuf.at[slot], o_hbm.at[pl.ds(row, _C)], osem)

    def wait_out():
        pltpu.make_async_copy(obuf.at[0], o_hbm.at[pl.ds(0, _C)], osem).wait()

    def compute(k):
        slot = lax.rem(k, _NBUF)

        @plsc.parallel_loop(0, _C * 128, 1, unroll=4)
        def _(q):
            rc = lax.shift_right_logical(q, 7)
            gg = q & 127
            g2 = lax.shift_right_logical(gg, 3) * 2
            col = (gg & 7) * 16
            a0 = gbuf[slot, rc, g2, pl.ds(col, 16)]
            a1 = gbuf[slot, rc, g2 + 1, pl.ds(col, 16)]
            u0 = ubuf[slot, rc, g2, pl.ds(col, 16)]
            u1 = ubuf[slot, rc, g2 + 1, pl.ds(col, 16)]
            l0 = _ln_poly(u0, _P8)
            l1 = _ln_poly(u1, _P8)
            lr = _ln_poly(l0 / l1, _P5)
            s = (a1 - a0 + lr) * (1.0 / TEMP)
            obuf[slot, rc, pl.ds(gg * 16, 16)] = 1.0 / (1.0 + jnp.exp(s))

    issue_in(0)

    def body(k, c):
        @pl.when(k + 1 < nchunks)
        def _():
            issue_in(k + 1)

        wait_in()

        @pl.when(k >= _NBUF)
        def _():
            wait_out()

        compute(k)
        issue_out(k)
        return c

    lax.fori_loop(0, nchunks, body, 0)
    for _ in range(min(nchunks, _NBUF)):
        wait_out()


def _sc_call(gm, uu, nrows):
    body = functools.partial(_sc_body, nrows)
    mesh = plsc.VectorSubcoreMesh(core_axis_name="c", subcore_axis_name="s")
    fn = pl.kernel(
        body,
        out_type=jax.ShapeDtypeStruct((nrows, SZ), jnp.float32),
        mesh=mesh,
        scratch_types=[
            pltpu.VMEM((_NBUF, _C, 32, 128), jnp.float32),
            pltpu.VMEM((_NBUF, _C, 32, 128), jnp.float32),
            pltpu.VMEM((_NBUF, _C, SZ), jnp.float32),
            pltpu.SemaphoreType.DMA,
            pltpu.SemaphoreType.DMA,
            pltpu.SemaphoreType.DMA,
        ],
    )
    return fn(gm, uu)


def kernel(gen_matrix, u):
    gm = _native_view(gen_matrix.reshape(SZ, SZ, 2))
    uu = _native_view(u.reshape(SZ, SZ, 2))
    if SC_ROWS == 0:
        return _tc_call(gm, uu, 0, SZ)
    if SC_ROWS == SZ:
        return _sc_call(gm, uu, SZ)
    top = _sc_call(gm, uu, SC_ROWS)
    bot = _tc_call(gm, uu, SC_ROWS, SZ - SC_ROWS, full=True)
    return lax.dynamic_update_slice(bot, top, (0, 0))


# speedup vs baseline: 1.1173x; 1.1042x over previous
import functools

import jax
import jax.numpy as jnp
from jax import lax
from jax.experimental import pallas as pl
from jax.experimental.pallas import tpu as pltpu
from jax.experimental.pallas import tpu_sc as plsc

SZ = 2048
TEMP = 10.0
EPS = 1e-20
LN2 = 0.6931471805599453

SC_ROWS = 896
TC_ROWS_PER_BLOCK = 64

_NC, _NS = 2, 16
_NW = _NC * _NS


def _native_view(x):
    return (
        x.reshape(SZ, 16, 128, 2)
        .transpose(0, 1, 3, 2)
        .reshape(SZ, 32, 128)
    )


def _tc_body(g_ref, u_ref, o_ref):
    for g in range(16):
        a0 = g_ref[:, 2 * g, :]
        a1 = g_ref[:, 2 * g + 1, :]
        u0 = u_ref[:, 2 * g, :]
        u1 = u_ref[:, 2 * g + 1, :]
        L0 = EPS - jnp.log(u0 + EPS)
        L1 = EPS - jnp.log(u1 + EPS)
        lr = jnp.log(L0 / L1)
        s = (a0 - a1 - lr) * (1.0 / TEMP)
        o_ref[:, 128 * g:128 * (g + 1)] = 1.0 / (1.0 + jnp.exp(-s))


def _tc_call(gm, uu, row0, nrows, full=False):
    grid = nrows // TC_ROWS_PER_BLOCK
    off = row0 // TC_ROWS_PER_BLOCK
    oshape = (SZ, SZ) if full else (nrows, SZ)
    oidx = (lambda i: (i + off, 0)) if full else (lambda i: (i, 0))
    return pl.pallas_call(
        _tc_body,
        grid=(grid,),
        in_specs=[
            pl.BlockSpec((TC_ROWS_PER_BLOCK, 32, 128), lambda i: (i + off, 0, 0)),
            pl.BlockSpec((TC_ROWS_PER_BLOCK, 32, 128), lambda i: (i + off, 0, 0)),
        ],
        out_specs=pl.BlockSpec((TC_ROWS_PER_BLOCK, SZ), oidx),
        out_shape=jax.ShapeDtypeStruct(oshape, jnp.float32),
    )(gm, uu)


_C = 4
_NBUF = 2


_P8 = (0.9999936302584942, -0.49982549864297415, 0.33144665223895153,
       -0.23943337072679435, 0.16499812978978992, -0.09229041732432854,
       0.03426459991863622, -0.006006605041074868)
_P5 = (0.9991664010110767, -0.48969909032090764, 0.2838231830655365,
       -0.12995719765851482, 0.029808765243560027)


def _ln_poly(v, coeffs):
    b = lax.bitcast_convert_type(v, jnp.int32)
    e = lax.shift_right_arithmetic(b, 23) - 127
    m = lax.bitcast_convert_type(
        (b & 0x007FFFFF) | 0x3F800000, jnp.float32)
    t = m - 1.0
    p = coeffs[-1]
    for c in reversed(coeffs[:-1]):
        p = c + t * p
    return e.astype(jnp.float32) * LN2 + t * p


def _sc_body(nrows, g_hbm, u_hbm, o_hbm, gbuf, ubuf, obuf, gsem, usem, osem):
    wid = lax.axis_index("s") * _NC + lax.axis_index("c")
    rpw = nrows // _NW
    base = wid * rpw
    nchunks = rpw // _C

    def issue_in(k):
        slot = lax.rem(k, _NBUF)
        row = base + k * _C
        pltpu.async_copy(g_hbm.at[pl.ds(row, _C)], gbuf.at[slot], gsem)
        pltpu.async_copy(u_hbm.at[pl.ds(row, _C)], ubuf.at[slot], usem)

    def wait_in():
        pltpu.make_async_copy(g_hbm.at[pl.ds(0, _C)], gbuf.at[0], gsem).wait()
        pltpu.make_async_copy(u_hbm.at[pl.ds(0, _C)], ubuf.at[0], usem).wait()

    def issue_out(k):
        slot = lax.rem(k, _NBUF)
        row = base + k * _C
        pltpu.async_copy(obuf.at[slot], o_hbm.at[pl.ds(row, _C)], osem)

    def wait_out():
        pltpu.make_async_copy(obuf.at[0], o_hbm.at[pl.ds(0, _C)], osem).wait()

    def compute(k):
        slot = lax.rem(k, _NBUF)

        @plsc.parallel_loop(0, _C * 128, 1, unroll=4)
        def _(q):
            rc = lax.shift_right_logical(q, 7)
            gg = q & 127
            g2 = lax.shift_right_logical(gg, 3) * 2
            col = (gg & 7) * 16
            a0 = gbuf[slot, rc, g2, pl.ds(col, 16)]
            a1 = gbuf[slot, rc, g2 + 1, pl.ds(col, 16)]
            u0 = ubuf[slot, rc, g2, pl.ds(col, 16)]
            u1 = ubuf[slot, rc, g2 + 1, pl.ds(col, 16)]
            l0 = _ln_poly(u0, _P8)
            l1 = _ln_poly(u1, _P8)
            lr = _ln_poly(l0 / l1, _P5)
            s = (a1 - a0 + lr) * (1.0 / TEMP)
            obuf[slot, rc, pl.ds(gg * 16, 16)] = 1.0 / (1.0 + jnp.exp(s))

    issue_in(0)

    def body(k, c):
        @pl.when(k + 1 < nchunks)
        def _():
            issue_in(k + 1)

        wait_in()

        @pl.when(k >= _NBUF)
        def _():
            wait_out()

        compute(k)
        issue_out(k)
        return c

    lax.fori_loop(0, nchunks, body, 0)
    for _ in range(min(nchunks, _NBUF)):
        wait_out()


def _sc_call(gm, uu, nrows):
    body = functools.partial(_sc_body, nrows)
    mesh = plsc.VectorSubcoreMesh(core_axis_name="c", subcore_axis_name="s")
    fn = pl.kernel(
        body,
        out_type=jax.ShapeDtypeStruct((nrows, SZ), jnp.float32),
        mesh=mesh,
        scratch_types=[
            pltpu.VMEM((_NBUF, _C, 32, 128), jnp.float32),
            pltpu.VMEM((_NBUF, _C, 32, 128), jnp.float32),
            pltpu.VMEM((_NBUF, _C, SZ), jnp.float32),
            pltpu.SemaphoreType.DMA,
            pltpu.SemaphoreType.DMA,
            pltpu.SemaphoreType.DMA,
        ],
    )
    return fn(gm, uu)


def kernel(gen_matrix, u):
    gm = _native_view(gen_matrix.reshape(SZ, SZ, 2))
    uu = _native_view(u.reshape(SZ, SZ, 2))
    if SC_ROWS == 0:
        return _tc_call(gm, uu, 0, SZ)
    if SC_ROWS == SZ:
        return _sc_call(gm, uu, SZ)
    top = _sc_call(gm, uu, SC_ROWS)
    bot = _tc_call(gm, uu, SC_ROWS, SZ - SC_ROWS, full=True)
    return lax.dynamic_update_slice(bot, top, (0, 0))


# speedup vs baseline: 1.2138x; 1.0864x over previous
import functools

import jax
import jax.numpy as jnp
from jax import lax
from jax.experimental import pallas as pl
from jax.experimental.pallas import tpu as pltpu
from jax.experimental.pallas import tpu_sc as plsc

SZ = 2048
TEMP = 10.0
EPS = 1e-20
LN2 = 0.6931471805599453

SC_ROWS = 896
TC_ROWS_PER_BLOCK = 64

_NC, _NS = 2, 16
_NW = _NC * _NS


def _native_view(x):
    return (
        x.reshape(SZ, 16, 128, 2)
        .transpose(0, 1, 3, 2)
        .reshape(SZ, 32, 128)
    )


def _tc_body(g_ref, u_ref, o_ref):
    for g in range(16):
        a0 = g_ref[:, 2 * g, :]
        a1 = g_ref[:, 2 * g + 1, :]
        u0 = u_ref[:, 2 * g, :]
        u1 = u_ref[:, 2 * g + 1, :]
        L0 = EPS - jnp.log(u0 + EPS)
        L1 = EPS - jnp.log(u1 + EPS)
        lr = jnp.log(L0 / L1)
        s = (a0 - a1 - lr) * (0.5 / TEMP)
        o_ref[:, 128 * g:128 * (g + 1)] = 0.5 + 0.5 * jnp.tanh(s)


def _tc_call(gm, uu, row0, nrows, full=False):
    grid = nrows // TC_ROWS_PER_BLOCK
    off = row0 // TC_ROWS_PER_BLOCK
    oshape = (SZ, SZ) if full else (nrows, SZ)
    oidx = (lambda i: (i + off, 0)) if full else (lambda i: (i, 0))
    return pl.pallas_call(
        _tc_body,
        grid=(grid,),
        in_specs=[
            pl.BlockSpec((TC_ROWS_PER_BLOCK, 32, 128), lambda i: (i + off, 0, 0)),
            pl.BlockSpec((TC_ROWS_PER_BLOCK, 32, 128), lambda i: (i + off, 0, 0)),
        ],
        out_specs=pl.BlockSpec((TC_ROWS_PER_BLOCK, SZ), oidx),
        out_shape=jax.ShapeDtypeStruct(oshape, jnp.float32),
    )(gm, uu)


_C = 4
_NBUF = 2


_P8 = (0.9999936302584942, -0.49982549864297415, 0.33144665223895153,
       -0.23943337072679435, 0.16499812978978992, -0.09229041732432854,
       0.03426459991863622, -0.006006605041074868)
_P5 = (0.9991664010110767, -0.48969909032090764, 0.2838231830655365,
       -0.12995719765851482, 0.029808765243560027)


def _ln_poly(v, coeffs):
    b = lax.bitcast_convert_type(v, jnp.int32)
    e = lax.shift_right_arithmetic(b, 23) - 127
    m = lax.bitcast_convert_type(
        (b & 0x007FFFFF) | 0x3F800000, jnp.float32)
    t = m - 1.0
    p = coeffs[-1]
    for c in reversed(coeffs[:-1]):
        p = c + t * p
    return e.astype(jnp.float32) * LN2 + t * p


def _sc_body(nrows, g_hbm, u_hbm, o_hbm, gbuf, ubuf, obuf, gsem, usem, osem):
    wid = lax.axis_index("s") * _NC + lax.axis_index("c")
    rpw = nrows // _NW
    base = wid * rpw
    nchunks = rpw // _C

    def issue_in(k):
        slot = lax.rem(k, _NBUF)
        row = base + k * _C
        pltpu.async_copy(g_hbm.at[pl.ds(row, _C)], gbuf.at[slot], gsem)
        pltpu.async_copy(u_hbm.at[pl.ds(row, _C)], ubuf.at[slot], usem)

    def wait_in():
        pltpu.make_async_copy(g_hbm.at[pl.ds(0, _C)], gbuf.at[0], gsem).wait()
        pltpu.make_async_copy(u_hbm.at[pl.ds(0, _C)], ubuf.at[0], usem).wait()

    def issue_out(k):
        slot = lax.rem(k, _NBUF)
        row = base + k * _C
        pltpu.async_copy(obuf.at[slot], o_hbm.at[pl.ds(row, _C)], osem)

    def wait_out():
        pltpu.make_async_copy(obuf.at[0], o_hbm.at[pl.ds(0, _C)], osem).wait()

    def compute(k):
        slot = lax.rem(k, _NBUF)

        @plsc.parallel_loop(0, _C * 128, 1, unroll=4)
        def _(q):
            rc = lax.shift_right_logical(q, 7)
            gg = q & 127
            g2 = lax.shift_right_logical(gg, 3) * 2
            col = (gg & 7) * 16
            a0 = gbuf[slot, rc, g2, pl.ds(col, 16)]
            a1 = gbuf[slot, rc, g2 + 1, pl.ds(col, 16)]
            u0 = ubuf[slot, rc, g2, pl.ds(col, 16)]
            u1 = ubuf[slot, rc, g2 + 1, pl.ds(col, 16)]
            l0 = _ln_poly(u0, _P8)
            l1 = _ln_poly(u1, _P8)
            lr = _ln_poly(l0 / l1, _P5)
            s = (a1 - a0 + lr) * (1.0 / TEMP)
            obuf[slot, rc, pl.ds(gg * 16, 16)] = 1.0 / (1.0 + jnp.exp(s))

    issue_in(0)

    def body(k, c):
        @pl.when(k + 1 < nchunks)
        def _():
            issue_in(k + 1)

        wait_in()

        @pl.when(k >= _NBUF)
        def _():
            wait_out()

        compute(k)
        issue_out(k)
        return c

    lax.fori_loop(0, nchunks, body, 0)
    for _ in range(min(nchunks, _NBUF)):
        wait_out()


def _sc_call(gm, uu, nrows):
    body = functools.partial(_sc_body, nrows)
    mesh = plsc.VectorSubcoreMesh(core_axis_name="c", subcore_axis_name="s")
    fn = pl.kernel(
        body,
        out_type=jax.ShapeDtypeStruct((nrows, SZ), jnp.float32),
        mesh=mesh,
        scratch_types=[
            pltpu.VMEM((_NBUF, _C, 32, 128), jnp.float32),
            pltpu.VMEM((_NBUF, _C, 32, 128), jnp.float32),
            pltpu.VMEM((_NBUF, _C, SZ), jnp.float32),
            pltpu.SemaphoreType.DMA,
            pltpu.SemaphoreType.DMA,
            pltpu.SemaphoreType.DMA,
        ],
    )
    return fn(gm, uu)


def kernel(gen_matrix, u):
    gm = _native_view(gen_matrix.reshape(SZ, SZ, 2))
    uu = _native_view(u.reshape(SZ, SZ, 2))
    if SC_ROWS == 0:
        return _tc_call(gm, uu, 0, SZ)
    if SC_ROWS == SZ:
        return _sc_call(gm, uu, SZ)
    top = _sc_call(gm, uu, SC_ROWS)
    bot = _tc_call(gm, uu, SC_ROWS, SZ - SC_ROWS, full=True)
    return lax.dynamic_update_slice(bot, top, (0, 0))
